# baseline (speedup 1.0000x reference)
import jax
import jax.numpy as jnp
from jax import lax
from jax.experimental import pallas as pl
from jax.experimental.pallas import tpu as pltpu
from jax.experimental.pallas import tpu_sc as plsc

_NC = 2
_NS = 16
_NW = _NC * _NS

_BATCH = 16384
_HIST = 50
_D = 64
_BP = _BATCH // _NW
_CT = _D // 8
_BT = _BATCH // 128
_BTW = _BP // 128


def _gather_body(idx_hbm, table_hbm, out_hbm, idx_v, rows_v, trans_v,
                 sg0, sg1, sw0, sw1):
    sg = (sg0, sg1)
    sw = (sw0, sw1)
    wid = lax.axis_index("s") * _NC + lax.axis_index("c")
    bbase = wid * _BP
    iota16 = lax.iota(jnp.int32, 16)

    pltpu.sync_copy(idx_hbm.at[:, pl.ds(bbase, _BP)], idx_v)

    def g_desc(h, b):
        return pltpu.make_async_copy(
            table_hbm.at[idx_v.at[h]], rows_v.at[b], sg[b])

    def w_desc(h, ct, p):
        return pltpu.make_async_copy(
            trans_v.at[p],
            out_hbm.at[h, ct, pl.ds(wid * _BTW, _BTW)], sw[p])

    def transpose_ct(b, ct, p):
        @pl.loop(0, 256, unroll=8)
        def _t(i):
            btl = i >> 6
            ci = (i >> 3) & 7
            bg = i & 7
            row = btl * 128 + bg * 16 + iota16
            col = jnp.full((16,), 8 * ct + ci, jnp.int32)
            val = plsc.load_gather(rows_v.at[b], [row, col])
            trans_v[p, btl, ci, pl.ds(bg * 16, 16)] = val

    g_desc(0, 0).start()
    g_desc(0, 0).wait()
    g_desc(1, 1).start()
    for ct in range(_CT):
        p = ct & 1
        if ct >= 2:
            w_desc(0, ct, p).wait()
        transpose_ct(0, ct, p)
        w_desc(0, ct, p).start()

    @pl.loop(0, (_HIST - 2) // 2)
    def _grp(i):
        for db in range(2):
            h = 2 * i + 1 + db
            b = (1 + db) & 1
            g_desc(h, b).wait()
            g_desc(h + 1, 1 - b).start()
            for ct in range(_CT):
                p = ct & 1
                w_desc(h, ct, p).wait()
                transpose_ct(b, ct, p)
                w_desc(h, ct, p).start()

    hl = _HIST - 1
    g_desc(hl, 1).wait()
    for ct in range(_CT):
        p = ct & 1
        w_desc(hl, ct, p).wait()
        transpose_ct(1, ct, p)
        w_desc(hl, ct, p).start()
    w_desc(hl, _CT - 2, 0).wait()
    w_desc(hl, _CT - 1, 1).wait()


_gather = pl.kernel(
    _gather_body,
    out_type=jax.ShapeDtypeStruct((_HIST, _CT, _BT, 8, 128), jnp.float32),
    mesh=plsc.VectorSubcoreMesh(core_axis_name="c", subcore_axis_name="s"),
    scratch_types=[
        pltpu.VMEM((_HIST, _BP), jnp.int32),
        pltpu.VMEM((2, _BP, _D), jnp.float32),
        pltpu.VMEM((2, _BTW, 8, 128), jnp.float32),
    ] + [pltpu.SemaphoreType.DMA] * 4,
    compiler_params=pltpu.CompilerParams(
        use_tc_tiling_on_sc=False, needs_layout_passes=False),
)


def kernel(token_ids, weights):
    ids_hm = jnp.swapaxes(token_ids, 0, 1)
    out5 = _gather(ids_hm, weights)
    t1 = jnp.transpose(out5, (0, 1, 3, 2, 4))
    r = jnp.reshape(t1, (_HIST, _D, _BATCH))
    return jnp.transpose(r, (2, 0, 1))


# speedup vs baseline: 1.4473x; 1.4473x over previous
import jax
import jax.numpy as jnp
from jax import lax
from jax.experimental import pallas as pl
from jax.experimental.pallas import tpu as pltpu
from jax.experimental.pallas import tpu_sc as plsc

_NC = 2
_NS = 16
_NW = _NC * _NS

_BATCH = 16384
_HIST = 50
_D = 64
_BP = _BATCH // _NW
_CT = _D // 8
_BT = _BATCH // 128
_BTW = _BP // 128


def _gather_body(idx_hbm, table_hbm, out_hbm, idx_v, rows_v, trans_v,
                 sg0, sg1, sw0, sw1):
    sg = (sg0, sg1)
    sw = (sw0, sw1)
    wid = lax.axis_index("s") * _NC + lax.axis_index("c")
    bbase = wid * _BP
    iota16 = lax.iota(jnp.int32, 16)

    pltpu.sync_copy(idx_hbm.at[:, pl.ds(bbase, _BP)], idx_v)

    def g_desc(h, b):
        return pltpu.make_async_copy(
            table_hbm.at[idx_v.at[h]], rows_v.at[b], sg[b])

    def w_desc(h, ct, p):
        return pltpu.make_async_copy(
            trans_v.at[p],
            out_hbm.at[h, ct, pl.ds(wid * _BTW, _BTW)], sw[p])

    civec = tuple(bg * 16 + iota16 for bg in range(8))

    def transpose_ct(b, ct, p):
        @plsc.parallel_loop(0, 32, unroll=2)
        def _t(i):
            btl = i >> 3
            ci = i & 7
            rbase = btl * 128
            col = jnp.full((16,), 8 * ct + ci, jnp.int32)
            for bg in range(8):
                row = rbase + civec[bg]
                val = plsc.load_gather(rows_v.at[b], [row, col])
                trans_v[p, btl, ci, pl.ds(bg * 16, 16)] = val

    g_desc(0, 0).start()
    g_desc(0, 0).wait()
    g_desc(1, 1).start()
    for ct in range(_CT):
        p = ct & 1
        if ct >= 2:
            w_desc(0, ct, p).wait()
        transpose_ct(0, ct, p)
        w_desc(0, ct, p).start()

    @pl.loop(0, (_HIST - 2) // 2)
    def _grp(i):
        for db in range(2):
            h = 2 * i + 1 + db
            b = (1 + db) & 1
            g_desc(h, b).wait()
            g_desc(h + 1, 1 - b).start()
            for ct in range(_CT):
                p = ct & 1
                w_desc(h, ct, p).wait()
                transpose_ct(b, ct, p)
                w_desc(h, ct, p).start()

    hl = _HIST - 1
    g_desc(hl, 1).wait()
    for ct in range(_CT):
        p = ct & 1
        w_desc(hl, ct, p).wait()
        transpose_ct(1, ct, p)
        w_desc(hl, ct, p).start()
    w_desc(hl, _CT - 2, 0).wait()
    w_desc(hl, _CT - 1, 1).wait()


_gather = pl.kernel(
    _gather_body,
    out_type=jax.ShapeDtypeStruct((_HIST, _CT, _BT, 8, 128), jnp.float32),
    mesh=plsc.VectorSubcoreMesh(core_axis_name="c", subcore_axis_name="s"),
    scratch_types=[
        pltpu.VMEM((_HIST, _BP), jnp.int32),
        pltpu.VMEM((2, _BP, _D), jnp.float32),
        pltpu.VMEM((2, _BTW, 8, 128), jnp.float32),
    ] + [pltpu.SemaphoreType.DMA] * 4,
    compiler_params=pltpu.CompilerParams(
        use_tc_tiling_on_sc=False, needs_layout_passes=False,
        disable_bounds_checks=True),
)


def kernel(token_ids, weights):
    ids_hm = jnp.swapaxes(token_ids, 0, 1)
    out5 = _gather(ids_hm, weights)
    t1 = jnp.transpose(out5, (0, 1, 3, 2, 4))
    r = jnp.reshape(t1, (_HIST, _D, _BATCH))
    return jnp.transpose(r, (2, 0, 1))
